# baseline (speedup 1.0000x reference)
import functools

import jax
import jax.numpy as jnp
from jax import lax
from jax.experimental import pallas as pl
from jax.experimental.pallas import tpu as pltpu
from jax.experimental.pallas import tpu_sc as plsc

_NSLOT = 4


def _gather_kernel(B, D, NC, NW, b_per_w, CH):
    n_ch = b_per_w // CH
    mesh = plsc.VectorSubcoreMesh(core_axis_name="c", subcore_axis_name="s")

    scratch = [pltpu.VMEM((b_per_w,), jnp.int32)]
    scratch += [pltpu.VMEM((CH, D), jnp.float32) for _ in range(2 * _NSLOT)]
    scratch += [pltpu.SemaphoreType.DMA for _ in range(4 * _NSLOT)]

    @functools.partial(
        pl.kernel,
        mesh=mesh,
        out_type=[
            jax.ShapeDtypeStruct((B, D), jnp.float32),
            jax.ShapeDtypeStruct((B, D), jnp.float32),
        ],
        scratch_types=scratch,
    )
    def k(ids_hbm, s_hbm, t_hbm, zs_hbm, zt_hbm, idx_v, *bufs):
        rows_s = bufs[0:_NSLOT]
        rows_t = bufs[_NSLOT:2 * _NSLOT]
        sems = bufs[2 * _NSLOT:]
        sem_gs = sems[0:_NSLOT]
        sem_gt = sems[_NSLOT:2 * _NSLOT]
        sem_ws = sems[2 * _NSLOT:3 * _NSLOT]
        sem_wt = sems[3 * _NSLOT:]

        wid = lax.axis_index("s") * NC + lax.axis_index("c")
        base = wid * b_per_w
        pltpu.sync_copy(ids_hbm.at[pl.ds(base, b_per_w)], idx_v)

        def start_gather(c):
            slot = c % _NSLOT
            idx_c = idx_v.at[pl.ds(c * CH, CH)]
            gs = pltpu.async_copy(s_hbm.at[idx_c], rows_s[slot], sem_gs[slot])
            gt = pltpu.async_copy(t_hbm.at[idx_c], rows_t[slot], sem_gt[slot])
            return gs, gt

        gathers = [None] * n_ch
        writes = [None] * n_ch
        gathers[0] = start_gather(0)
        for c in range(n_ch):
            slot = c % _NSLOT
            if c + 1 < n_ch:
                if c + 1 >= _NSLOT:
                    ws, wt = writes[c + 1 - _NSLOT]
                    ws.wait()
                    wt.wait()
                gathers[c + 1] = start_gather(c + 1)
            gs, gt = gathers[c]
            gs.wait()
            gt.wait()
            dst = pl.ds(base + c * CH, CH)
            ws = pltpu.async_copy(rows_s[slot], zs_hbm.at[dst], sem_ws[slot])
            wt = pltpu.async_copy(rows_t[slot], zt_hbm.at[dst], sem_wt[slot])
            writes[c] = (ws, wt)
        for c in range(max(0, n_ch - _NSLOT), n_ch):
            ws, wt = writes[c]
            ws.wait()
            wt.wait()

    return k


def kernel(object_ids, shape_table, texture_table):
    B = object_ids.shape[0]
    D = shape_table.shape[1]
    info = plsc.get_sparse_core_info()
    NC, NS = info.num_cores, info.num_subcores
    NW = NC * NS
    b_per_w = B // NW
    CH = 64

    ids = object_ids.astype(jnp.int32)
    k = _gather_kernel(B, D, NC, NW, b_per_w, CH)
    z_s, z_t = k(ids, shape_table, texture_table)
    return (z_s, z_t)


# speedup vs baseline: 1.0125x; 1.0125x over previous
import functools

import jax
import jax.numpy as jnp
from jax import lax
from jax.experimental import pallas as pl
from jax.experimental.pallas import tpu as pltpu
from jax.experimental.pallas import tpu_sc as plsc

_NSLOT = 3


def _gather_kernel(B, D, NC, NW, b_per_w, CH):
    n_ch = b_per_w // CH
    mesh = plsc.VectorSubcoreMesh(core_axis_name="c", subcore_axis_name="s")

    scratch = [pltpu.VMEM((b_per_w,), jnp.int32)]
    scratch += [pltpu.VMEM((CH, D), jnp.float32) for _ in range(_NSLOT)]
    scratch += [pltpu.SemaphoreType.DMA for _ in range(2 * _NSLOT)]

    @functools.partial(
        pl.kernel,
        mesh=mesh,
        out_type=[
            jax.ShapeDtypeStruct((B, D), jnp.float32),
            jax.ShapeDtypeStruct((B, D), jnp.float32),
        ],
        scratch_types=scratch,
    )
    def k(ids_hbm, s_hbm, t_hbm, zs_hbm, zt_hbm, idx_v, *bufs):
        rows = bufs[0:_NSLOT]
        sem_g = bufs[_NSLOT:2 * _NSLOT]
        sem_w = bufs[2 * _NSLOT:]

        wid = lax.axis_index("s") * NC + lax.axis_index("c")
        pair = wid // 2
        base = pair * b_per_w
        pltpu.sync_copy(ids_hbm.at[pl.ds(base, b_per_w)], idx_v)

        def body(tbl_hbm, out_hbm):
            def start_gather(c):
                slot = c % _NSLOT
                idx_c = idx_v.at[pl.ds(c * CH, CH)]
                return pltpu.async_copy(tbl_hbm.at[idx_c], rows[slot], sem_g[slot])

            gathers = [None] * n_ch
            writes = [None] * n_ch
            gathers[0] = start_gather(0)
            for c in range(n_ch):
                slot = c % _NSLOT
                if c + 1 < n_ch:
                    if c + 1 >= _NSLOT:
                        writes[c + 1 - _NSLOT].wait()
                    gathers[c + 1] = start_gather(c + 1)
                gathers[c].wait()
                dst = pl.ds(base + c * CH, CH)
                writes[c] = pltpu.async_copy(rows[slot], out_hbm.at[dst], sem_w[slot])
            for c in range(max(0, n_ch - _NSLOT), n_ch):
                writes[c].wait()

        @pl.when(wid % 2 == 0)
        def _():
            body(s_hbm, zs_hbm)

        @pl.when(wid % 2 == 1)
        def _():
            body(t_hbm, zt_hbm)

    return k


def kernel(object_ids, shape_table, texture_table):
    B = object_ids.shape[0]
    D = shape_table.shape[1]
    info = plsc.get_sparse_core_info()
    NC, NS = info.num_cores, info.num_subcores
    NW = NC * NS
    b_per_w = (2 * B) // NW
    CH = 128

    ids = object_ids.astype(jnp.int32)
    k = _gather_kernel(B, D, NC, NW, b_per_w, CH)
    z_s, z_t = k(ids, shape_table, texture_table)
    return (z_s, z_t)


# speedup vs baseline: 1.0210x; 1.0083x over previous
import functools

import jax
import jax.numpy as jnp
from jax import lax
from jax.experimental import pallas as pl
from jax.experimental.pallas import tpu as pltpu
from jax.experimental.pallas import tpu_sc as plsc

_NSLOT = 3


def _gather_kernel(B, D, NC, NW, b_per_w, CH):
    n_ch = b_per_w // CH
    mesh = plsc.VectorSubcoreMesh(core_axis_name="c", subcore_axis_name="s")

    scratch = [pltpu.VMEM((b_per_w,), jnp.int32)]
    scratch += [pltpu.VMEM((CH, D), jnp.float32) for _ in range(_NSLOT)]
    scratch += [pltpu.SemaphoreType.DMA for _ in range(2 * _NSLOT)]

    @functools.partial(
        pl.kernel,
        mesh=mesh,
        out_type=[
            jax.ShapeDtypeStruct((B, D), jnp.float32),
            jax.ShapeDtypeStruct((B, D), jnp.float32),
        ],
        scratch_types=scratch,
    )
    def k(ids_hbm, s_hbm, t_hbm, zs_hbm, zt_hbm, idx_v, *bufs):
        rows = bufs[0:_NSLOT]
        sem_g = bufs[_NSLOT:2 * _NSLOT]
        sem_w = bufs[2 * _NSLOT:]

        wid = lax.axis_index("s") * NC + lax.axis_index("c")
        pair = wid // 2
        base = pair * b_per_w
        pltpu.sync_copy(ids_hbm.at[pl.ds(base, b_per_w)], idx_v)

        def body(tbl_hbm, out_hbm):
            def start_gather(c):
                slot = c % _NSLOT
                idx_c = idx_v.at[pl.ds(c * CH, CH)]
                return pltpu.async_copy(tbl_hbm.at[idx_c], rows[slot], sem_g[slot])

            gathers = [None] * n_ch
            writes = [None] * n_ch
            gathers[0] = start_gather(0)
            for c in range(n_ch):
                slot = c % _NSLOT
                if c + 1 < n_ch:
                    if c + 1 >= _NSLOT:
                        writes[c + 1 - _NSLOT].wait()
                    gathers[c + 1] = start_gather(c + 1)
                gathers[c].wait()
                dst = pl.ds(base + c * CH, CH)
                writes[c] = pltpu.async_copy(rows[slot], out_hbm.at[dst], sem_w[slot])
            for c in range(max(0, n_ch - _NSLOT), n_ch):
                writes[c].wait()

        @pl.when(wid % 2 == 0)
        def _():
            body(s_hbm, zs_hbm)

        @pl.when(wid % 2 == 1)
        def _():
            body(t_hbm, zt_hbm)

    return k


def kernel(object_ids, shape_table, texture_table):
    B = object_ids.shape[0]
    D = shape_table.shape[1]
    info = plsc.get_sparse_core_info()
    NC, NS = info.num_cores, info.num_subcores
    NW = NC * NS
    b_per_w = (2 * B) // NW
    CH = 256

    ids = object_ids.astype(jnp.int32)
    k = _gather_kernel(B, D, NC, NW, b_per_w, CH)
    z_s, z_t = k(ids, shape_table, texture_table)
    return (z_s, z_t)
